# baseline (speedup 1.0000x reference)
import functools

import jax
import jax.numpy as jnp
from jax import lax
from jax.experimental import pallas as pl
from jax.experimental.pallas import tpu as pltpu
from jax.experimental.pallas import tpu_sc as plsc

N_NODES = 10000
D = 128
N_EDGES = 320000
NC, NS = 2, 16
NW = NC * NS
GROUP = 128
G = -(-N_EDGES // (NW * GROUP))
EDGES_PER_WORKER = G * GROUP
E_PAD = NW * EDGES_PER_WORKER
ROWS_PER_TILE = 632
N_PAD = NS * ROWS_PER_TILE
PAD_SRC = N_NODES

_mesh = plsc.VectorSubcoreMesh(
    core_axis_name="c", subcore_axis_name="s", num_cores=NC, num_subcores=NS
)


@functools.partial(
    pl.kernel,
    out_type=jax.ShapeDtypeStruct((NC, N_PAD, D), jnp.float32),
    mesh=_mesh,
    scratch_types=[
        pltpu.VMEM((G, GROUP), jnp.int32),
        pltpu.VMEM((G, GROUP), jnp.int32),
        pltpu.VMEM((GROUP, D), jnp.float32),
        pltpu.VMEM_SHARED((N_PAD, D), jnp.float32),
        pltpu.SemaphoreType.DMA,
    ],
)
def _gcn_aggregate(feat_hbm, src_hbm, dst_hbm, zeros_hbm, out_hbm,
                   idx_s, idx_d, rows, hagg, sem):
    cid = lax.axis_index("c")
    sid = lax.axis_index("s")
    wid = sid * NC + cid

    with jax.named_scope("agg_init"):
        pltpu.sync_copy(
            zeros_hbm.at[pl.ds(sid * ROWS_PER_TILE, ROWS_PER_TILE)],
            hagg.at[pl.ds(sid * ROWS_PER_TILE, ROWS_PER_TILE)])
        pltpu.sync_copy(src_hbm.at[wid], idx_s)
        pltpu.sync_copy(dst_hbm.at[wid], idx_d)
        plsc.subcore_barrier()

    with jax.named_scope("agg_edges"):
        def body(j, carry):
            pltpu.async_copy(feat_hbm.at[idx_s.at[j]], rows, sem).wait()
            pltpu.sync_copy(rows, hagg.at[idx_d.at[j]], add=True)
            return carry

        lax.fori_loop(0, G, body, 0)
        plsc.subcore_barrier()

    with jax.named_scope("agg_writeout"):
        pltpu.sync_copy(
            hagg.at[pl.ds(sid * ROWS_PER_TILE, ROWS_PER_TILE)],
            out_hbm.at[cid, pl.ds(sid * ROWS_PER_TILE, ROWS_PER_TILE)])


def _linear_relu_body(parts_ref, wt_ref, b_ref, o_ref):
    x = parts_ref[0] + parts_ref[1]
    y = jnp.dot(x, wt_ref[...], preferred_element_type=jnp.float32)
    o_ref[...] = jnp.maximum(y + b_ref[...], 0.0)


_BLK = N_PAD // 8


def _apply_linear_relu(parts, wt, b2):
    return pl.pallas_call(
        _linear_relu_body,
        grid=(N_PAD // _BLK,),
        in_specs=[
            pl.BlockSpec((NC, _BLK, D), lambda i: (0, i, 0)),
            pl.BlockSpec((D, D), lambda i: (0, 0)),
            pl.BlockSpec((1, D), lambda i: (0, 0)),
        ],
        out_specs=pl.BlockSpec((_BLK, D), lambda i: (i, 0)),
        out_shape=jax.ShapeDtypeStruct((N_PAD, D), jnp.float32),
    )(parts, wt, b2)


@jax.jit
def kernel(feature, edge_index, W, b):
    src = edge_index[0].astype(jnp.int32)
    dst = edge_index[1].astype(jnp.int32)
    pad = E_PAD - N_EDGES
    src_p = jnp.concatenate(
        [src, jnp.full((pad,), PAD_SRC, jnp.int32)]).reshape(NW, G, GROUP)
    dst_p = jnp.concatenate(
        [dst, jnp.zeros((pad,), jnp.int32)]).reshape(NW, G, GROUP)
    feat_ext = jnp.concatenate(
        [feature, jnp.zeros((16, D), feature.dtype)], axis=0)
    zeros = jnp.zeros((N_PAD, D), jnp.float32)
    parts = _gcn_aggregate(feat_ext, src_p, dst_p, zeros)
    return _apply_linear_relu(parts, W.T, b.reshape(1, D))[:N_NODES]


# speedup vs baseline: 1.1097x; 1.1097x over previous
import functools

import jax
import jax.numpy as jnp
from jax import lax
from jax.experimental import pallas as pl
from jax.experimental.pallas import tpu as pltpu
from jax.experimental.pallas import tpu_sc as plsc

N_NODES = 10000
D = 128
N_EDGES = 320000
NC, NS = 2, 16
NW = NC * NS
GROUP = 128
G = -(-N_EDGES // (NW * GROUP))
EDGES_PER_WORKER = G * GROUP
E_PAD = NW * EDGES_PER_WORKER
ROWS_PER_TILE = 632
N_PAD = NS * ROWS_PER_TILE
PAD_SRC = N_NODES

_mesh = plsc.VectorSubcoreMesh(
    core_axis_name="c", subcore_axis_name="s", num_cores=NC, num_subcores=NS
)


@functools.partial(
    pl.kernel,
    out_type=jax.ShapeDtypeStruct((NC, N_PAD, D), jnp.float32),
    mesh=_mesh,
    scratch_types=[
        pltpu.VMEM((G, GROUP), jnp.int32),
        pltpu.VMEM((G, GROUP), jnp.int32),
        pltpu.VMEM((GROUP, D), jnp.float32),
        pltpu.VMEM_SHARED((N_PAD, D), jnp.float32),
        pltpu.SemaphoreType.DMA,
    ],
)
def _gcn_aggregate(feat_hbm, feat2_hbm, src_hbm, dst_hbm, zeros_hbm, out_hbm,
                   idx_s, idx_d, rows, hagg, sem):
    cid = lax.axis_index("c")
    sid = lax.axis_index("s")
    wid = sid * NC + cid

    with jax.named_scope("agg_init"):
        pltpu.sync_copy(
            zeros_hbm.at[pl.ds(sid * ROWS_PER_TILE, ROWS_PER_TILE)],
            hagg.at[pl.ds(sid * ROWS_PER_TILE, ROWS_PER_TILE)])
        pltpu.sync_copy(src_hbm.at[wid], idx_s)
        pltpu.sync_copy(dst_hbm.at[wid], idx_d)
        plsc.subcore_barrier()

    with jax.named_scope("agg_edges"):
        def run(feat_ref):
            def body(j, carry):
                pltpu.async_copy(feat_ref.at[idx_s.at[j]], rows, sem).wait()
                pltpu.sync_copy(rows, hagg.at[idx_d.at[j]], add=True)
                return carry

            lax.fori_loop(0, G, body, 0)

        @pl.when(cid == 0)
        def _():
            run(feat_hbm)

        @pl.when(cid == 1)
        def _():
            run(feat2_hbm)

        plsc.subcore_barrier()

    with jax.named_scope("agg_writeout"):
        pltpu.sync_copy(
            hagg.at[pl.ds(sid * ROWS_PER_TILE, ROWS_PER_TILE)],
            out_hbm.at[cid, pl.ds(sid * ROWS_PER_TILE, ROWS_PER_TILE)])


def _linear_relu_body(parts_ref, wt_ref, b_ref, o_ref):
    x = parts_ref[0] + parts_ref[1]
    y = jnp.dot(x, wt_ref[...], preferred_element_type=jnp.float32)
    o_ref[...] = jnp.maximum(y + b_ref[...], 0.0)


_BLK = N_PAD // 8


def _apply_linear_relu(parts, wt, b2):
    return pl.pallas_call(
        _linear_relu_body,
        grid=(N_PAD // _BLK,),
        in_specs=[
            pl.BlockSpec((NC, _BLK, D), lambda i: (0, i, 0)),
            pl.BlockSpec((D, D), lambda i: (0, 0)),
            pl.BlockSpec((1, D), lambda i: (0, 0)),
        ],
        out_specs=pl.BlockSpec((_BLK, D), lambda i: (i, 0)),
        out_shape=jax.ShapeDtypeStruct((N_PAD, D), jnp.float32),
    )(parts, wt, b2)


@jax.jit
def kernel(feature, edge_index, W, b):
    src = edge_index[0].astype(jnp.int32)
    dst = edge_index[1].astype(jnp.int32)
    pad = E_PAD - N_EDGES
    src_p = jnp.concatenate(
        [src, jnp.full((pad,), PAD_SRC, jnp.int32)]).reshape(NW, G, GROUP)
    dst_p = jnp.concatenate(
        [dst, jnp.zeros((pad,), jnp.int32)]).reshape(NW, G, GROUP)
    feat_ext = jnp.concatenate(
        [feature, jnp.zeros((16, D), feature.dtype)], axis=0)
    feat_ext2 = jnp.concatenate(
        [lax.optimization_barrier(feature), jnp.zeros((16, D), feature.dtype)],
        axis=0)
    zeros = jnp.zeros((N_PAD, D), jnp.float32)
    parts = _gcn_aggregate(feat_ext, feat_ext2, src_p, dst_p, zeros)
    return _apply_linear_relu(parts, W.T, b.reshape(1, D))[:N_NODES]
